# baseline (speedup 1.0000x reference)
import functools

import jax
import jax.numpy as jnp
from jax import lax
from jax.experimental import pallas as pl
from jax.experimental.pallas import tpu as pltpu

_T = 8192
_D = 2048
_E = 64
_TOP_K = 8
_N_GROUPS = 8
_EPG = _E // _N_GROUPS
_TOPK_GROUPS = 4
_SCALE = 2.5

_BLK_T = 2048
_NEG = -1e30


def _router_block(x_ref, w_ref, b_ref, wts_ref, idx_ref):
    x = x_ref[...]
    w = w_ref[...]
    scores = jax.nn.sigmoid(jnp.dot(x, w, preferred_element_type=jnp.float32))
    st = jnp.transpose(scores)
    bt = st + b_ref[...]
    B = st.shape[1]

    st3 = jnp.reshape(st, (_N_GROUPS, _EPG, B))
    bt3 = jnp.reshape(bt, (_N_GROUPS, _EPG, B))
    riota = lax.broadcasted_iota(jnp.int32, (_EPG, B), 0)

    gsums = []
    for g in range(_N_GROUPS):
        sg = bt3[g]
        m1 = jnp.max(sg, axis=0, keepdims=True)
        i1 = jnp.min(jnp.where(sg == m1, riota, _EPG), axis=0, keepdims=True)
        m2 = jnp.max(jnp.where(riota == i1, _NEG, sg), axis=0, keepdims=True)
        gsums.append(m1 + m2)

    gs = jnp.concatenate(gsums, axis=0)
    giota = lax.broadcasted_iota(jnp.int32, (_N_GROUPS, B), 0)
    keep = jnp.zeros((_N_GROUPS, B), dtype=jnp.bool_)
    for _ in range(_TOPK_GROUPS):
        best = jnp.max(gs, axis=0, keepdims=True)
        cg = jnp.min(jnp.where(gs == best, giota, _N_GROUPS), axis=0,
                     keepdims=True)
        keep = keep | (giota == cg)
        gs = jnp.where(giota == cg, _NEG, gs)

    eidx = (lax.broadcasted_iota(jnp.int32, (_N_GROUPS, _EPG, B), 0) * _EPG
            + lax.broadcasted_iota(jnp.int32, (_N_GROUPS, _EPG, B), 1))
    keep3 = jnp.reshape(keep, (_N_GROUPS, 1, B))
    masked = jnp.where(keep3, bt3, 0.0)

    w_rows = []
    i_rows = []
    for _ in range(_TOP_K):
        m8 = jnp.max(masked, axis=1)
        m = jnp.max(m8, axis=0, keepdims=True)
        is_m = masked == jnp.reshape(m, (1, 1, B))
        ik8 = jnp.min(jnp.where(is_m, eidx, _E), axis=1)
        ik = jnp.min(ik8, axis=0, keepdims=True)
        sel = eidx == jnp.reshape(ik, (1, 1, B))
        wk8 = jnp.max(jnp.where(sel, st3, _NEG), axis=1)
        wk = jnp.max(wk8, axis=0, keepdims=True)
        masked = jnp.where(sel, _NEG, masked)
        w_rows.append(wk)
        i_rows.append(ik)

    wt = jnp.concatenate(w_rows, axis=0)
    it = jnp.concatenate(i_rows, axis=0)
    s = jnp.sum(wt, axis=0, keepdims=True)
    wt = wt * (_SCALE / (s + 1e-20))

    wts_ref[...] = jnp.transpose(wt)
    idx_ref[...] = jnp.transpose(it)


@jax.jit
def kernel(x, kernel_DE, bias_E):
    x = jnp.asarray(x, jnp.float32)
    bias_2d = jnp.reshape(bias_E, (_E, 1))
    grid = (_T // _BLK_T,)
    wts, idx = pl.pallas_call(
        _router_block,
        grid=grid,
        in_specs=[
            pl.BlockSpec((_BLK_T, _D), lambda i: (i, 0)),
            pl.BlockSpec((_D, _E), lambda i: (0, 0)),
            pl.BlockSpec((_E, 1), lambda i: (0, 0)),
        ],
        out_specs=[
            pl.BlockSpec((_BLK_T, _TOP_K), lambda i: (i, 0)),
            pl.BlockSpec((_BLK_T, _TOP_K), lambda i: (i, 0)),
        ],
        out_shape=[
            jax.ShapeDtypeStruct((_T, _TOP_K), jnp.float32),
            jax.ShapeDtypeStruct((_T, _TOP_K), jnp.int32),
        ],
    )(x, kernel_DE, bias_2d)
    return (wts, idx)


# speedup vs baseline: 1.0286x; 1.0286x over previous
import functools

import jax
import jax.numpy as jnp
from jax import lax
from jax.experimental import pallas as pl
from jax.experimental.pallas import tpu as pltpu

_T = 8192
_D = 2048
_E = 64
_TOP_K = 8
_N_GROUPS = 8
_EPG = _E // _N_GROUPS
_TOPK_GROUPS = 4
_SCALE = 2.5

_BLK_T = 1024
_HD = _D // 2
_NEG = -1e30


def _router_block(xa_ref, xb_ref, w_ref, b_ref, wts_ref, idx_ref):
    w = w_ref[...]
    acc = jnp.dot(xa_ref[...], w[:_HD], preferred_element_type=jnp.float32)
    acc = acc + jnp.dot(xb_ref[...], w[_HD:], preferred_element_type=jnp.float32)
    scores = jax.nn.sigmoid(acc)
    st = jnp.transpose(scores)
    bt = st + b_ref[...]
    B = st.shape[1]

    st3 = jnp.reshape(st, (_N_GROUPS, _EPG, B))
    bt3 = jnp.reshape(bt, (_N_GROUPS, _EPG, B))
    riota = lax.broadcasted_iota(jnp.int32, (_EPG, B), 0)

    gsums = []
    for g in range(_N_GROUPS):
        sg = bt3[g]
        m1 = jnp.max(sg, axis=0, keepdims=True)
        i1 = jnp.min(jnp.where(sg == m1, riota, _EPG), axis=0, keepdims=True)
        m2 = jnp.max(jnp.where(riota == i1, _NEG, sg), axis=0, keepdims=True)
        gsums.append(m1 + m2)

    gs = jnp.concatenate(gsums, axis=0)
    giota = lax.broadcasted_iota(jnp.int32, (_N_GROUPS, B), 0)
    keep = jnp.zeros((_N_GROUPS, B), dtype=jnp.bool_)
    for _ in range(_TOPK_GROUPS):
        best = jnp.max(gs, axis=0, keepdims=True)
        cg = jnp.min(jnp.where(gs == best, giota, _N_GROUPS), axis=0,
                     keepdims=True)
        keep = keep | (giota == cg)
        gs = jnp.where(giota == cg, _NEG, gs)

    eidx = (lax.broadcasted_iota(jnp.int32, (_N_GROUPS, _EPG, B), 0) * _EPG
            + lax.broadcasted_iota(jnp.int32, (_N_GROUPS, _EPG, B), 1))
    keep3 = jnp.reshape(keep, (_N_GROUPS, 1, B))
    masked = jnp.where(keep3, bt3, 0.0)

    w_rows = []
    i_rows = []
    for _ in range(_TOP_K):
        m8 = jnp.max(masked, axis=1)
        m = jnp.max(m8, axis=0, keepdims=True)
        is_m = masked == jnp.reshape(m, (1, 1, B))
        ik8 = jnp.min(jnp.where(is_m, eidx, _E), axis=1)
        ik = jnp.min(ik8, axis=0, keepdims=True)
        sel = eidx == jnp.reshape(ik, (1, 1, B))
        wk8 = jnp.max(jnp.where(sel, st3, _NEG), axis=1)
        wk = jnp.max(wk8, axis=0, keepdims=True)
        masked = jnp.where(sel, _NEG, masked)
        w_rows.append(wk)
        i_rows.append(ik)

    wt = jnp.concatenate(w_rows, axis=0)
    it = jnp.concatenate(i_rows, axis=0)
    s = jnp.sum(wt, axis=0, keepdims=True)
    wt = wt * (_SCALE / (s + 1e-20))

    wts_ref[...] = jnp.transpose(wt)
    idx_ref[...] = jnp.transpose(it)


@jax.jit
def kernel(x, kernel_DE, bias_E):
    x = jnp.asarray(x, jnp.float32)
    bias_2d = jnp.reshape(bias_E, (_E, 1))
    grid = (_T // _BLK_T,)
    wts, idx = pl.pallas_call(
        _router_block,
        grid=grid,
        in_specs=[
            pl.BlockSpec((_BLK_T, _HD), lambda i: (i, 0)),
            pl.BlockSpec((_BLK_T, _HD), lambda i: (i, 1)),
            pl.BlockSpec((_D, _E), lambda i: (0, 0)),
            pl.BlockSpec((_E, 1), lambda i: (0, 0)),
        ],
        out_specs=[
            pl.BlockSpec((_BLK_T, _TOP_K), lambda i: (i, 0)),
            pl.BlockSpec((_BLK_T, _TOP_K), lambda i: (i, 0)),
        ],
        out_shape=[
            jax.ShapeDtypeStruct((_T, _TOP_K), jnp.float32),
            jax.ShapeDtypeStruct((_T, _TOP_K), jnp.int32),
        ],
    )(x, x, kernel_DE, bias_2d)
    return (wts, idx)
